# baseline (speedup 1.0000x reference)
import functools

import jax
import jax.numpy as jnp
from jax import lax
from jax.experimental import pallas as pl
from jax.experimental.pallas import tpu as pltpu
from jax.experimental.pallas import tpu_sc as plsc

_LOW = 0.4
_HIGH = 0.75
_P = 128
_B1 = 20000
_B1P = 20480
_NC = _B1P // 128
_BIG = 2 ** 30


def _select_kernel(ibp_ref, tbp_ref, pos_ref, neg_ref, ptg_ref,
                   pb_ref, nb_ref, imax_ref, iidx_ref, ps_ref, ns_ref,
                   psT_ref, nsT_ref):
    lane = lax.broadcasted_iota(jnp.int32, (1, 128), 1)
    tgt_iota = lax.broadcasted_iota(jnp.int32, (128, 1), 0)

    tcx = tbp_ref[0]
    tcy = tbp_ref[1]
    tw = tbp_ref[2]
    th = tbp_ref[3]
    tx0 = tcx - tw * 0.5
    ty0 = tcy - th * 0.5
    tx1 = tcx + tw * 0.5
    ty1 = tcy + th * 0.5
    area_t = jnp.maximum(tx1 - tx0, 0.0) * jnp.maximum(ty1 - ty0, 0.0)

    def body1(c, carry):
        colmax, colidx = carry
        icx = ibp_ref[0, pl.ds(c, 1), :]
        icy = ibp_ref[1, pl.ds(c, 1), :]
        iw = ibp_ref[2, pl.ds(c, 1), :]
        ih = ibp_ref[3, pl.ds(c, 1), :]
        ix0 = icx - iw * 0.5
        iy0 = icy - ih * 0.5
        ix1 = icx + iw * 0.5
        iy1 = icy + ih * 0.5
        area_i = jnp.maximum(ix1 - ix0, 0.0) * jnp.maximum(iy1 - iy0, 0.0)
        x0 = jnp.maximum(ix0, tx0)
        y0 = jnp.maximum(iy0, ty0)
        x1 = jnp.minimum(ix1, tx1)
        y1 = jnp.minimum(iy1, ty1)
        inter = jnp.maximum(x1 - x0, 0.0) * jnp.maximum(y1 - y0, 0.0)
        union = area_i + area_t - inter
        iou = inter / jnp.maximum(union, 1e-8)
        im = jnp.max(iou, axis=0, keepdims=True)
        ia = jnp.min(jnp.where(iou == im, tgt_iota, jnp.int32(128)),
                     axis=0, keepdims=True)
        imax_ref[pl.ds(c, 1), :] = im
        iidx_ref[pl.ds(c, 1), :] = ia
        cm = jnp.max(iou, axis=1, keepdims=True)
        gidx = c * 128 + lane
        ca = jnp.min(jnp.where(iou == cm, gidx, _BIG), axis=1, keepdims=True)
        upd = cm > colmax
        return jnp.where(upd, cm, colmax), jnp.where(upd, ca, colidx)

    colmax0 = jnp.full((128, 1), -1.0, jnp.float32)
    colidx0 = jnp.zeros((128, 1), jnp.int32)
    _, colidx = lax.fori_loop(0, _NC, body1, (colmax0, colidx0))

    def body2(c, _):
        gidx = c * 128 + lane
        im = imax_ref[pl.ds(c, 1), :]
        mem = jnp.any(colidx == gidx, axis=0, keepdims=True)
        posm = (im >= _HIGH) | mem
        negm = (im < _LOW) & jnp.logical_not(posm)
        valid = gidx < _B1
        ps = jnp.where(valid, jnp.where(posm, im, -1.0), -2.0)
        ns = jnp.where(valid, jnp.where(negm, 1.0 - im, -1.0), -2.0)
        ps_ref[pl.ds(c, 1), :] = ps
        ns_ref[pl.ds(c, 1), :] = ns
        return 0

    lax.fori_loop(0, _NC, body2, 0)

    citer = lax.broadcasted_iota(jnp.int32, (1, _NC), 1)
    payT = (citer * 128
            + lax.broadcasted_iota(jnp.int32, (128, _NC), 0))
    s8 = lax.broadcasted_iota(jnp.int32, (8, 16), 0)
    l16 = lax.broadcasted_iota(jnp.int32, (8, 16), 1)

    psT = ps_ref[...].T
    psT_ref[...] = psT
    nsT = ns_ref[...].T
    nsT_ref[...] = nsT
    rp = jnp.max(psT, axis=1, keepdims=True)
    pp = jnp.min(jnp.where(psT == rp, payT, _BIG), axis=1, keepdims=True)
    rn = jnp.max(nsT, axis=1, keepdims=True)
    pn = jnp.min(jnp.where(nsT == rn, payT, _BIG), axis=1, keepdims=True)
    rp = rp.reshape(8, 16)
    pp = pp.reshape(8, 16)
    rn = rn.reshape(8, 16)
    pn = pn.reshape(8, 16)

    def _extract(sref, rmax, rpay):
        m = jnp.max(rmax)
        wpay = jnp.min(jnp.where(rmax == m, rpay, _BIG))
        s = wpay % 128
        ch = wpay // 128
        rowv = sref[pl.ds(s, 1), :]
        newrow = jnp.where(citer == ch, -3.0, rowv)
        sref[pl.ds(s, 1), :] = newrow
        nm = jnp.max(newrow)
        npay = jnp.min(jnp.where(newrow == nm, citer * 128 + s, _BIG))
        upd = (s8 == s // 16) & (l16 == s % 16)
        rmax = jnp.where(upd, nm, rmax)
        rpay = jnp.where(upd, npay, rpay)
        return wpay, s, ch, rmax, rpay

    def body3(i, carry):
        pacc, nacc, tacc, pbox, nbox, rp, pp, rn, pn = carry
        sel = lane == i
        w, s, ch, rp, pp = _extract(psT_ref, rp, pp)
        lane_eq = lane == s
        tv = jnp.max(jnp.where(lane_eq, iidx_ref[pl.ds(ch, 1), :],
                               jnp.int32(-1)))
        pbox = [jnp.where(sel,
                          jnp.max(jnp.where(lane_eq,
                                            ibp_ref[k, pl.ds(ch, 1), :],
                                            -1e30)), pbox[k])
                for k in range(4)]
        w2, s2, ch2, rn, pn = _extract(nsT_ref, rn, pn)
        lane_eq2 = lane == s2
        nbox = [jnp.where(sel,
                          jnp.max(jnp.where(lane_eq2,
                                            ibp_ref[k, pl.ds(ch2, 1), :],
                                            -1e30)), nbox[k])
                for k in range(4)]
        return (jnp.where(sel, w, pacc), jnp.where(sel, w2, nacc),
                jnp.where(sel, tv, tacc), pbox, nbox, rp, pp, rn, pn)

    z = jnp.zeros((1, 128), jnp.int32)
    zf = [jnp.zeros((1, 128), jnp.float32) for _ in range(4)]
    pacc, nacc, tacc, pbox, nbox, rp, pp, rn, pn = lax.fori_loop(
        0, _P, body3, (z, z, z, zf, zf, rp, pp, rn, pn))
    pos_ref[...] = pacc
    neg_ref[...] = nacc
    ptg_ref[...] = tacc
    for k in range(4):
        pb_ref[pl.ds(k, 1), :] = pbox[k]
        nb_ref[pl.ds(k, 1), :] = nbox[k]


def _select(ibp, tbp):
    return pl.pallas_call(
        _select_kernel,
        out_shape=[jax.ShapeDtypeStruct((1, 128), jnp.int32)] * 3
        + [jax.ShapeDtypeStruct((4, 128), jnp.float32)] * 2,
        scratch_shapes=[
            pltpu.VMEM((_NC, 128), jnp.float32),
            pltpu.VMEM((_NC, 128), jnp.int32),
            pltpu.VMEM((_NC, 128), jnp.float32),
            pltpu.VMEM((_NC, 128), jnp.float32),
            pltpu.VMEM((128, _NC), jnp.float32),
            pltpu.VMEM((128, _NC), jnp.float32),
        ],
    )(ibp, tbp)


def _gather_body(feats, ttab, pos_idx, ptg_idx, pos_data_o, tgt_o,
                 idx8, rfeat, rtgt, sem):
    wid = lax.axis_index("s") * 2 + lax.axis_index("c")

    @pl.when(wid < 16)
    def _():
        base = wid * 8
        pltpu.sync_copy(pos_idx.at[pl.ds(base, 8)], idx8)
        pltpu.async_copy(feats.at[idx8], rfeat, sem).wait()
        pltpu.sync_copy(rfeat, pos_data_o.at[pl.ds(base, 8)])

    @pl.when(wid >= 16)
    def _():
        base = (wid - 16) * 8
        pltpu.sync_copy(ptg_idx.at[pl.ds(base, 8)], idx8)
        pltpu.async_copy(ttab.at[idx8], rtgt, sem).wait()
        pltpu.sync_copy(rtgt, tgt_o.at[pl.ds(base, 8)])


def _gather_kernel(inf, ttab, pos_i, ptg_i):
    mesh = plsc.VectorSubcoreMesh(core_axis_name="c", subcore_axis_name="s")
    k = pl.kernel(
        _gather_body,
        mesh=mesh,
        out_type=[
            jax.ShapeDtypeStruct((128, 256), jnp.float32),
            jax.ShapeDtypeStruct((128, 128), jnp.float32),
        ],
        scratch_types=[
            pltpu.VMEM((8,), jnp.int32),
            pltpu.VMEM((8, 256), jnp.float32),
            pltpu.VMEM((8, 128), jnp.float32),
            pltpu.SemaphoreType.DMA,
        ],
    )
    return k(inf, ttab, pos_i, ptg_i)


@jax.jit
def kernel(input_boxes, input_feats, target_boxes, target_feats):
    ib = input_boxes[0]
    tb = target_boxes[0]
    inf = input_feats[0]
    tgf = target_feats[0]
    ibp = jnp.pad(ib, ((0, _B1P - _B1), (0, 0))).T.reshape(4, _NC, 128)
    tbp = tb.T.reshape(4, 128, 1)
    pos, neg, ptg, pb, nb = _select(ibp, tbp)
    pos_i = pos.reshape(128)
    ptg_i = ptg.reshape(128)
    ttab = jnp.pad(jnp.concatenate([tb, tgf], axis=1), ((0, 0), (0, 60)))
    pos_d, tgt_rows = _gather_kernel(inf, ttab, pos_i, ptg_i)
    return (pb.T, pos_d, tgt_rows[:, :4], tgt_rows[:, 4:68], nb.T)


# speedup vs baseline: 1.0421x; 1.0421x over previous
import functools

import jax
import jax.numpy as jnp
from jax import lax
from jax.experimental import pallas as pl
from jax.experimental.pallas import tpu as pltpu
from jax.experimental.pallas import tpu_sc as plsc

_LOW = 0.4
_HIGH = 0.75
_P = 128
_B1 = 20000
_B1P = 20480
_NC = _B1P // 128
_BIG = 2 ** 30


def _select_kernel(ibp_ref, tbp_ref, pos_ref, neg_ref, ptg_ref,
                   pb_ref, nb_ref, imax_ref, iidx_ref, ps_ref, ns_ref,
                   psT_ref, nsT_ref):
    lane = lax.broadcasted_iota(jnp.int32, (1, 128), 1)
    tgt_iota = lax.broadcasted_iota(jnp.int32, (128, 1), 0)

    tcx = tbp_ref[0]
    tcy = tbp_ref[1]
    tw = tbp_ref[2]
    th = tbp_ref[3]
    tx0 = tcx - tw * 0.5
    ty0 = tcy - th * 0.5
    tx1 = tcx + tw * 0.5
    ty1 = tcy + th * 0.5
    area_t = jnp.maximum(tx1 - tx0, 0.0) * jnp.maximum(ty1 - ty0, 0.0)

    def body1(c, carry):
        colmax, colidx = carry
        icx = ibp_ref[0, pl.ds(c, 1), :]
        icy = ibp_ref[1, pl.ds(c, 1), :]
        iw = ibp_ref[2, pl.ds(c, 1), :]
        ih = ibp_ref[3, pl.ds(c, 1), :]
        ix0 = icx - iw * 0.5
        iy0 = icy - ih * 0.5
        ix1 = icx + iw * 0.5
        iy1 = icy + ih * 0.5
        area_i = jnp.maximum(ix1 - ix0, 0.0) * jnp.maximum(iy1 - iy0, 0.0)
        x0 = jnp.maximum(ix0, tx0)
        y0 = jnp.maximum(iy0, ty0)
        x1 = jnp.minimum(ix1, tx1)
        y1 = jnp.minimum(iy1, ty1)
        inter = jnp.maximum(x1 - x0, 0.0) * jnp.maximum(y1 - y0, 0.0)
        union = area_i + area_t - inter
        iou = inter / jnp.maximum(union, 1e-8)
        im = jnp.max(iou, axis=0, keepdims=True)
        ia = jnp.min(jnp.where(iou == im, tgt_iota, jnp.int32(128)),
                     axis=0, keepdims=True)
        imax_ref[pl.ds(c, 1), :] = im
        iidx_ref[pl.ds(c, 1), :] = ia
        cm = jnp.max(iou, axis=1, keepdims=True)
        gidx = c * 128 + lane
        ca = jnp.min(jnp.where(iou == cm, gidx, _BIG), axis=1, keepdims=True)
        upd = cm > colmax
        return jnp.where(upd, cm, colmax), jnp.where(upd, ca, colidx)

    colmax0 = jnp.full((128, 1), -1.0, jnp.float32)
    colidx0 = jnp.zeros((128, 1), jnp.int32)
    _, colidx = lax.fori_loop(0, _NC, body1, (colmax0, colidx0))

    def body2(c, _):
        gidx = c * 128 + lane
        im = imax_ref[pl.ds(c, 1), :]
        mem = jnp.any(colidx == gidx, axis=0, keepdims=True)
        posm = (im >= _HIGH) | mem
        negm = (im < _LOW) & jnp.logical_not(posm)
        valid = gidx < _B1
        ps = jnp.where(valid, jnp.where(posm, im, -1.0), -2.0)
        ns = jnp.where(valid, jnp.where(negm, 1.0 - im, -1.0), -2.0)
        ps_ref[pl.ds(c, 1), :] = ps
        ns_ref[pl.ds(c, 1), :] = ns
        return 0

    lax.fori_loop(0, _NC, body2, 0)

    lane256 = lax.broadcasted_iota(jnp.int32, (1, 256), 1)
    payT = (lax.broadcasted_iota(jnp.int32, (128, 256), 1) * 128
            + lax.broadcasted_iota(jnp.int32, (128, 256), 0))

    def _treemax(sv, pv, width):
        n = width
        while n > 1:
            n //= 2
            s2 = jnp.roll(sv, -n, axis=1)
            p2 = jnp.roll(pv, -n, axis=1)
            take2 = (s2 > sv) | ((s2 == sv) & (p2 < pv))
            sv = jnp.where(take2, s2, sv)
            pv = jnp.where(take2, p2, pv)
        return sv, pv

    def _treemax1(sv, width):
        n = width
        while n > 1:
            n //= 2
            sv = jnp.maximum(sv, jnp.roll(sv, -n, axis=1))
        return sv

    psT = ps_ref[...].T
    psT_ref[:, : _NC] = psT
    psT_ref[:, _NC:] = jnp.full((128, 256 - _NC), -3.0, jnp.float32)
    nsT = ns_ref[...].T
    nsT_ref[:, : _NC] = nsT
    nsT_ref[:, _NC:] = jnp.full((128, 256 - _NC), -3.0, jnp.float32)
    psTf = psT_ref[...]
    nsTf = nsT_ref[...]
    rp0 = jnp.max(psTf, axis=1, keepdims=True)
    pp0 = jnp.min(jnp.where(psTf == rp0, payT, _BIG), axis=1, keepdims=True)
    rn0 = jnp.max(nsTf, axis=1, keepdims=True)
    pn0 = jnp.min(jnp.where(nsTf == rn0, payT, _BIG), axis=1, keepdims=True)
    rp = rp0.T
    pp = pp0.T
    rn = rn0.T
    pn = pn0.T

    def _extract(sref, rmax, rpay):
        wsv, wpv = _treemax(rmax, rpay, 128)
        wpay = wpv[0, 0]
        s = wpay % 128
        ch = wpay // 128
        rowv = sref[pl.ds(s, 1), :]
        newrow = jnp.where(lane256 == ch, -3.0, rowv)
        sref[pl.ds(s, 1), :] = newrow
        payrow = lane256 * 128 + s
        nsv, npv = _treemax(newrow, payrow, 256)
        su = lane == s
        rmax = jnp.where(su, nsv[:, :128], rmax)
        rpay = jnp.where(su, npv[:, :128], rpay)
        return wpv[:, :128], s, ch, rmax, rpay

    def body3(i, carry):
        pacc, nacc, tacc, pbox, nbox, rp, pp, rn, pn = carry
        sel = lane == i
        wv, s, ch, rp, pp = _extract(psT_ref, rp, pp)
        lane_eq = lane == s
        tvv = _treemax1(jnp.where(lane_eq, iidx_ref[pl.ds(ch, 1), :],
                                  jnp.int32(-1)), 128)
        pbox = [jnp.where(sel,
                          _treemax1(jnp.where(lane_eq,
                                              ibp_ref[k, pl.ds(ch, 1), :],
                                              -1e30), 128), pbox[k])
                for k in range(4)]
        wv2, s2, ch2, rn, pn = _extract(nsT_ref, rn, pn)
        lane_eq2 = lane == s2
        nbox = [jnp.where(sel,
                          _treemax1(jnp.where(lane_eq2,
                                              ibp_ref[k, pl.ds(ch2, 1), :],
                                              -1e30), 128), nbox[k])
                for k in range(4)]
        return (jnp.where(sel, wv, pacc), jnp.where(sel, wv2, nacc),
                jnp.where(sel, tvv, tacc), pbox, nbox, rp, pp, rn, pn)

    z = jnp.zeros((1, 128), jnp.int32)
    zf = [jnp.zeros((1, 128), jnp.float32) for _ in range(4)]
    pacc, nacc, tacc, pbox, nbox, rp, pp, rn, pn = lax.fori_loop(
        0, _P, body3, (z, z, z, zf, zf, rp, pp, rn, pn))
    pos_ref[...] = pacc
    neg_ref[...] = nacc
    ptg_ref[...] = tacc
    for k in range(4):
        pb_ref[pl.ds(k, 1), :] = pbox[k]
        nb_ref[pl.ds(k, 1), :] = nbox[k]


def _select(ibp, tbp):
    return pl.pallas_call(
        _select_kernel,
        out_shape=[jax.ShapeDtypeStruct((1, 128), jnp.int32)] * 3
        + [jax.ShapeDtypeStruct((4, 128), jnp.float32)] * 2,
        scratch_shapes=[
            pltpu.VMEM((_NC, 128), jnp.float32),
            pltpu.VMEM((_NC, 128), jnp.int32),
            pltpu.VMEM((_NC, 128), jnp.float32),
            pltpu.VMEM((_NC, 128), jnp.float32),
            pltpu.VMEM((128, 256), jnp.float32),
            pltpu.VMEM((128, 256), jnp.float32),
        ],
    )(ibp, tbp)


def _gather_body(feats, ttab, pos_idx, ptg_idx, pos_data_o, tgt_o,
                 idx8, rfeat, rtgt, sem):
    wid = lax.axis_index("s") * 2 + lax.axis_index("c")

    @pl.when(wid < 16)
    def _():
        base = wid * 8
        pltpu.sync_copy(pos_idx.at[pl.ds(base, 8)], idx8)
        pltpu.async_copy(feats.at[idx8], rfeat, sem).wait()
        pltpu.sync_copy(rfeat, pos_data_o.at[pl.ds(base, 8)])

    @pl.when(wid >= 16)
    def _():
        base = (wid - 16) * 8
        pltpu.sync_copy(ptg_idx.at[pl.ds(base, 8)], idx8)
        pltpu.async_copy(ttab.at[idx8], rtgt, sem).wait()
        pltpu.sync_copy(rtgt, tgt_o.at[pl.ds(base, 8)])


def _gather_kernel(inf, ttab, pos_i, ptg_i):
    mesh = plsc.VectorSubcoreMesh(core_axis_name="c", subcore_axis_name="s")
    k = pl.kernel(
        _gather_body,
        mesh=mesh,
        out_type=[
            jax.ShapeDtypeStruct((128, 256), jnp.float32),
            jax.ShapeDtypeStruct((128, 128), jnp.float32),
        ],
        scratch_types=[
            pltpu.VMEM((8,), jnp.int32),
            pltpu.VMEM((8, 256), jnp.float32),
            pltpu.VMEM((8, 128), jnp.float32),
            pltpu.SemaphoreType.DMA,
        ],
    )
    return k(inf, ttab, pos_i, ptg_i)


@jax.jit
def kernel(input_boxes, input_feats, target_boxes, target_feats):
    ib = input_boxes[0]
    tb = target_boxes[0]
    inf = input_feats[0]
    tgf = target_feats[0]
    ibp = jnp.pad(ib, ((0, _B1P - _B1), (0, 0))).T.reshape(4, _NC, 128)
    tbp = tb.T.reshape(4, 128, 1)
    pos, neg, ptg, pb, nb = _select(ibp, tbp)
    pos_i = pos.reshape(128)
    ptg_i = ptg.reshape(128)
    ttab = jnp.pad(jnp.concatenate([tb, tgf], axis=1), ((0, 0), (0, 60)))
    pos_d, tgt_rows = _gather_kernel(inf, ttab, pos_i, ptg_i)
    return (pb.T, pos_d, tgt_rows[:, :4], tgt_rows[:, 4:68], nb.T)


# speedup vs baseline: 1.1781x; 1.1305x over previous
import functools

import jax
import jax.numpy as jnp
from jax import lax
from jax.experimental import pallas as pl
from jax.experimental.pallas import tpu as pltpu
from jax.experimental.pallas import tpu_sc as plsc

_LOW = 0.4
_HIGH = 0.75
_P = 128
_B1 = 20000
_B1P = 20480
_NC = _B1P // 128
_BIG = 2 ** 30


def _select_kernel(ibp_ref, tbp_ref, pos_ref, neg_ref, ptg_ref,
                   pb_ref, nb_ref, imax_ref, iidx_ref, ps_ref, ns_ref):
    lane = lax.broadcasted_iota(jnp.int32, (1, 128), 1)
    tgt_iota = lax.broadcasted_iota(jnp.int32, (128, 1), 0)

    tcx = tbp_ref[0]
    tcy = tbp_ref[1]
    tw = tbp_ref[2]
    th = tbp_ref[3]
    tx0 = tcx - tw * 0.5
    ty0 = tcy - th * 0.5
    tx1 = tcx + tw * 0.5
    ty1 = tcy + th * 0.5
    area_t = jnp.maximum(tx1 - tx0, 0.0) * jnp.maximum(ty1 - ty0, 0.0)

    def body1(c, carry):
        colmax, colidx = carry
        icx = ibp_ref[0, pl.ds(c, 1), :]
        icy = ibp_ref[1, pl.ds(c, 1), :]
        iw = ibp_ref[2, pl.ds(c, 1), :]
        ih = ibp_ref[3, pl.ds(c, 1), :]
        ix0 = icx - iw * 0.5
        iy0 = icy - ih * 0.5
        ix1 = icx + iw * 0.5
        iy1 = icy + ih * 0.5
        area_i = jnp.maximum(ix1 - ix0, 0.0) * jnp.maximum(iy1 - iy0, 0.0)
        x0 = jnp.maximum(ix0, tx0)
        y0 = jnp.maximum(iy0, ty0)
        x1 = jnp.minimum(ix1, tx1)
        y1 = jnp.minimum(iy1, ty1)
        inter = jnp.maximum(x1 - x0, 0.0) * jnp.maximum(y1 - y0, 0.0)
        union = area_i + area_t - inter
        iou = inter / jnp.maximum(union, 1e-8)
        im = jnp.max(iou, axis=0, keepdims=True)
        ia = jnp.min(jnp.where(iou == im, tgt_iota, jnp.int32(128)),
                     axis=0, keepdims=True)
        imax_ref[pl.ds(c, 1), :] = im
        iidx_ref[pl.ds(c, 1), :] = ia
        cm = jnp.max(iou, axis=1, keepdims=True)
        gidx = c * 128 + lane
        ca = jnp.min(jnp.where(iou == cm, gidx, _BIG), axis=1, keepdims=True)
        upd = cm > colmax
        return jnp.where(upd, cm, colmax), jnp.where(upd, ca, colidx)

    colmax0 = jnp.full((128, 1), -1.0, jnp.float32)
    colidx0 = jnp.zeros((128, 1), jnp.int32)
    _, colidx = lax.fori_loop(0, _NC, body1, (colmax0, colidx0))

    def body2(c, _):
        gidx = c * 128 + lane
        im = imax_ref[pl.ds(c, 1), :]
        mem = jnp.any(colidx == gidx, axis=0, keepdims=True)
        posm = (im >= _HIGH) | mem
        negm = (im < _LOW) & jnp.logical_not(posm)
        valid = gidx < _B1
        ps = jnp.where(valid, jnp.where(posm, im, -1.0), -2.0)
        ns = jnp.where(valid, jnp.where(negm, 1.0 - im, -1.0), -2.0)
        ps_ref[pl.ds(c, 1), :] = ps
        ns_ref[pl.ds(c, 1), :] = ns
        return 0

    lax.fori_loop(0, _NC, body2, 0)

    gidx_all = (lax.broadcasted_iota(jnp.int32, (_NC, 128), 0) * 128
                + lax.broadcasted_iota(jnp.int32, (_NC, 128), 1))

    def body3(i, carry):
        pacc, nacc, tacc, pbox, nbox = carry
        sel = lane == i
        ps = ps_ref[...]
        m = jnp.max(ps)
        w = jnp.min(jnp.where(ps == m, gidx_all, _BIG))
        row = w // 128
        lane_eq = lane == (w - row * 128)
        tv = jnp.max(jnp.where(lane_eq, iidx_ref[pl.ds(row, 1), :],
                               jnp.int32(-1)))
        ps_ref[pl.ds(row, 1), :] = jnp.where(lane_eq, -3.0,
                                             ps_ref[pl.ds(row, 1), :])
        pbox = [jnp.where(sel,
                          jnp.max(jnp.where(lane_eq,
                                            ibp_ref[k, pl.ds(row, 1), :],
                                            -1e30)), pbox[k])
                for k in range(4)]
        ns = ns_ref[...]
        m2 = jnp.max(ns)
        w2 = jnp.min(jnp.where(ns == m2, gidx_all, _BIG))
        row2 = w2 // 128
        lane_eq2 = lane == (w2 - row2 * 128)
        ns_ref[pl.ds(row2, 1), :] = jnp.where(lane_eq2, -3.0,
                                              ns_ref[pl.ds(row2, 1), :])
        nbox = [jnp.where(sel,
                          jnp.max(jnp.where(lane_eq2,
                                            ibp_ref[k, pl.ds(row2, 1), :],
                                            -1e30)), nbox[k])
                for k in range(4)]
        return (jnp.where(sel, w, pacc), jnp.where(sel, w2, nacc),
                jnp.where(sel, tv, tacc), pbox, nbox)

    z = jnp.zeros((1, 128), jnp.int32)
    zf = [jnp.zeros((1, 128), jnp.float32) for _ in range(4)]
    pacc, nacc, tacc, pbox, nbox = lax.fori_loop(
        0, _P, body3, (z, z, z, zf, zf))
    pos_ref[...] = pacc
    neg_ref[...] = nacc
    ptg_ref[...] = tacc
    for k in range(4):
        pb_ref[pl.ds(k, 1), :] = pbox[k]
        nb_ref[pl.ds(k, 1), :] = nbox[k]


def _select(ibp, tbp):
    return pl.pallas_call(
        _select_kernel,
        out_shape=[jax.ShapeDtypeStruct((1, 128), jnp.int32)] * 3
        + [jax.ShapeDtypeStruct((4, 128), jnp.float32)] * 2,
        scratch_shapes=[
            pltpu.VMEM((_NC, 128), jnp.float32),
            pltpu.VMEM((_NC, 128), jnp.int32),
            pltpu.VMEM((_NC, 128), jnp.float32),
            pltpu.VMEM((_NC, 128), jnp.float32),
        ],
    )(ibp, tbp)


def _gather_body(feats, ttab, pos_idx, ptg_idx, pos_data_o, tgt_o,
                 idx8, rfeat, rtgt, sem):
    wid = lax.axis_index("s") * 2 + lax.axis_index("c")

    @pl.when(wid < 16)
    def _():
        base = wid * 8
        pltpu.sync_copy(pos_idx.at[pl.ds(base, 8)], idx8)
        pltpu.async_copy(feats.at[idx8], rfeat, sem).wait()
        pltpu.sync_copy(rfeat, pos_data_o.at[pl.ds(base, 8)])

    @pl.when(wid >= 16)
    def _():
        base = (wid - 16) * 8
        pltpu.sync_copy(ptg_idx.at[pl.ds(base, 8)], idx8)
        pltpu.async_copy(ttab.at[idx8], rtgt, sem).wait()
        pltpu.sync_copy(rtgt, tgt_o.at[pl.ds(base, 8)])


def _gather_kernel(inf, ttab, pos_i, ptg_i):
    mesh = plsc.VectorSubcoreMesh(core_axis_name="c", subcore_axis_name="s")
    k = pl.kernel(
        _gather_body,
        mesh=mesh,
        out_type=[
            jax.ShapeDtypeStruct((128, 256), jnp.float32),
            jax.ShapeDtypeStruct((128, 128), jnp.float32),
        ],
        scratch_types=[
            pltpu.VMEM((8,), jnp.int32),
            pltpu.VMEM((8, 256), jnp.float32),
            pltpu.VMEM((8, 128), jnp.float32),
            pltpu.SemaphoreType.DMA,
        ],
    )
    return k(inf, ttab, pos_i, ptg_i)


@jax.jit
def kernel(input_boxes, input_feats, target_boxes, target_feats):
    ib = input_boxes[0]
    tb = target_boxes[0]
    inf = input_feats[0]
    tgf = target_feats[0]
    ibp = jnp.pad(ib, ((0, _B1P - _B1), (0, 0))).T.reshape(4, _NC, 128)
    tbp = tb.T.reshape(4, 128, 1)
    pos, neg, ptg, pb, nb = _select(ibp, tbp)
    pos_i = pos.reshape(128)
    ptg_i = ptg.reshape(128)
    ttab = jnp.pad(jnp.concatenate([tb, tgf], axis=1), ((0, 0), (0, 60)))
    pos_d, tgt_rows = _gather_kernel(inf, ttab, pos_i, ptg_i)
    return (pb.T, pos_d, tgt_rows[:, :4], tgt_rows[:, 4:68], nb.T)
